# baseline (speedup 1.0000x reference)
import jax
import jax.numpy as jnp
from jax import lax
from jax.experimental import pallas as pl
from jax.experimental.pallas import tpu as pltpu
from jax.experimental.pallas import tpu_sc as plsc

K = 32
B = 16384
NF = 3
NC = 2
NS = 16
L = 16
NW = NC * NS
BPW = B // NW
CHUNK = 128
NCHUNK = BPW // CHUNK
NG = BPW // L


def _body(idx_hbm, f0_hbm, f1_hbm, f2_hbm, out_hbm,
          idx_v, r0, r1, r2, out_v, sem):
    wid = lax.axis_index("s") * NC + lax.axis_index("c")
    base = wid * BPW

    for i in range(NF):
        for j in range(NCHUNK):
            pltpu.sync_copy(idx_hbm.at[i, pl.ds(base + j * CHUNK, CHUNK)],
                            idx_v.at[i, j])

    copies = []
    for i, (f_hbm, r) in enumerate(((f0_hbm, r0), (f1_hbm, r1), (f2_hbm, r2))):
        for j in range(NCHUNK):
            copies.append(pltpu.async_copy(
                f_hbm.at[idx_v.at[i, j]],
                r.at[pl.ds(j * CHUNK, CHUNK)], sem))
    for c in copies:
        c.wait()

    iota = lax.iota(jnp.int32, L)

    def group(g, carry):
        rowv = g * L + iota
        acc = jnp.zeros((L,), jnp.float32)
        for kk in range(K):
            kv = jnp.full((L,), kk, jnp.int32)
            a = plsc.load_gather(r0, [rowv, kv])
            b = plsc.load_gather(r1, [rowv, kv])
            c = plsc.load_gather(r2, [rowv, kv])
            acc = acc + a * b * c
        out_v[pl.ds(g * L, L)] = acc
        return carry

    lax.fori_loop(0, NG, group, 0)
    pltpu.sync_copy(out_v, out_hbm.at[pl.ds(base, BPW)])


@jax.jit
def kernel(indices, factor0, factor1, factor2):
    idx = indices.astype(jnp.int32)
    mesh = plsc.VectorSubcoreMesh(core_axis_name="c", subcore_axis_name="s")
    f = pl.kernel(
        _body,
        out_type=jax.ShapeDtypeStruct((B,), jnp.float32),
        mesh=mesh,
        scratch_types=[
            pltpu.VMEM((NF, NCHUNK, CHUNK), jnp.int32),
            pltpu.VMEM((BPW, K), jnp.float32),
            pltpu.VMEM((BPW, K), jnp.float32),
            pltpu.VMEM((BPW, K), jnp.float32),
            pltpu.VMEM((BPW,), jnp.float32),
            pltpu.SemaphoreType.DMA,
        ],
        compiler_params=pltpu.CompilerParams(
            needs_layout_passes=False, use_tc_tiling_on_sc=False),
    )
    return f(idx, factor0, factor1, factor2)


# speedup vs baseline: 1.2629x; 1.2629x over previous
import jax
import jax.numpy as jnp
from jax import lax
from jax.experimental import pallas as pl
from jax.experimental.pallas import tpu as pltpu
from jax.experimental.pallas import tpu_sc as plsc

K = 32
KP = 128
B = 16384
NF = 3
NC = 2
NS = 16
L = 16
NW = NC * NS
BPW = B // NW
HALF = BPW // 2
NGH = HALF // L
DRAIN_ROWS = (NF * HALF * K) // KP


def _body(idx_hbm, f0_hbm, f1_hbm, f2_hbm, out_hbm,
          idx0, idx1, idx2, r0, r1, r2, out_v, dummy_hbm, sem):
    wid = lax.axis_index("s") * NC + lax.axis_index("c")
    base = wid * BPW

    for i, idxb in enumerate((idx0, idx1, idx2)):
        pltpu.sync_copy(idx_hbm.at[pl.ds(i * B + base, BPW)], idxb)

    iota = lax.iota(jnp.int32, L)

    for h in range(2):
        hbase = h * HALF

        def issue(g, carry):
            for idxb, t_hbm, r in ((idx0, f0_hbm, r0),
                                   (idx1, f1_hbm, r1),
                                   (idx2, f2_hbm, r2)):
                vec = idxb[pl.ds(hbase + g * L, L)]
                for jj in range(L):
                    rr = vec[jj]
                    pltpu.async_copy(
                        t_hbm.at[pl.ds(rr, 1), pl.ds(0, K)],
                        r.at[pl.ds(g * L + jj, 1), pl.ds(0, K)], sem)
            return carry

        lax.fori_loop(0, NGH, issue, 0)

        for r in (r0, r1, r2):
            pltpu.make_async_copy(dummy_hbm, r, sem).wait()

        def group(g, carry):
            rowv = g * L + iota
            acc = jnp.zeros((L,), jnp.float32)
            for kk in range(K):
                kv = jnp.full((L,), kk, jnp.int32)
                a = plsc.load_gather(r0, [rowv, kv])
                b = plsc.load_gather(r1, [rowv, kv])
                c = plsc.load_gather(r2, [rowv, kv])
                acc = acc + a * b * c
            out_v[pl.ds(hbase + g * L, L)] = acc
            return carry

        lax.fori_loop(0, NGH, group, 0)

    pltpu.sync_copy(out_v, out_hbm.at[pl.ds(base, BPW)])


@jax.jit
def kernel(indices, factor0, factor1, factor2):
    idx = indices.astype(jnp.int32).reshape(NF * B)
    mesh = plsc.VectorSubcoreMesh(core_axis_name="c", subcore_axis_name="s")
    f = pl.kernel(
        _body,
        out_type=jax.ShapeDtypeStruct((B,), jnp.float32),
        mesh=mesh,
        scratch_types=[
            pltpu.VMEM((BPW,), jnp.int32),
            pltpu.VMEM((BPW,), jnp.int32),
            pltpu.VMEM((BPW,), jnp.int32),
            pltpu.VMEM((HALF, K), jnp.float32),
            pltpu.VMEM((HALF, K), jnp.float32),
            pltpu.VMEM((HALF, K), jnp.float32),
            pltpu.VMEM((BPW,), jnp.float32),
            pltpu.MemorySpace.HBM((HALF, K), jnp.float32),
            pltpu.SemaphoreType.DMA,
        ],
        compiler_params=pltpu.CompilerParams(
            needs_layout_passes=False, use_tc_tiling_on_sc=True),
    )
    return f(idx, factor0, factor1, factor2)
